# Initial kernel scaffold; baseline (speedup 1.0000x reference)
import jax
import jax.numpy as jnp
from jax.experimental import pallas as pl


def kernel(x, edge_index, edge_type, Wq0, Wk0, Wv0, Et0, g0, b0, Wq1, Wk1, Wv1, Et1, g1, b1):
    raise NotImplementedError("write your pallas kernel here")


# speedup vs baseline: 13.7784x; 13.7784x over previous
import jax
import jax.numpy as jnp
from jax import lax
from jax.experimental import pallas as pl
from jax.experimental.pallas import tpu as pltpu
from jax.experimental.pallas import tpu_sc as plsc

N = 10000
E = 320000
D = 128
H = 8
DH = 16
NT = 16
NC = 2
NS = 16
NW = NC * NS
EPW = E // NW
C = 40
NCHUNK = EPW // C
ACC_W = 144
RPT = N // NS
ZR = 25


def _head_selector(shape_hd):
    if shape_hd == "dh":
        d = lax.broadcasted_iota(jnp.int32, (D, H), 0) // DH
        h = lax.broadcasted_iota(jnp.int32, (D, H), 1)
    else:
        h = lax.broadcasted_iota(jnp.int32, (H, D), 0)
        d = lax.broadcasted_iota(jnp.int32, (H, D), 1) // DH
    return (d == h).astype(jnp.float32)


def _tc_qkv_body(x_ref, wq_ref, wk_ref, wv_ref, e0_ref,
                 kv_ref, q_ref, sout_ref, ssum_ref):
    x = x_ref[...]
    q = jnp.dot(x, wq_ref[...], preferred_element_type=jnp.float32)
    k = jnp.dot(x, wk_ref[...], preferred_element_type=jnp.float32)
    v = jnp.dot(x, wv_ref[...], preferred_element_type=jnp.float32)
    q_ref[...] = q
    kv_ref[:, :D] = k
    kv_ref[:, D:] = v
    e0 = e0_ref[...]
    sel_dh = _head_selector("dh")
    a = jnp.dot(q * (k + e0), sel_dh, preferred_element_type=jnp.float32) * 0.25
    w = jnp.exp(a)
    ssum_ref[...] = w
    w128 = jnp.dot(w, _head_selector("hd"), preferred_element_type=jnp.float32)
    sout_ref[...] = w128 * (v + e0)


def _tc_combine_body(acc_ref, sout_ref, ssum_ref, g_ref, b_ref, o_ref):
    tot = acc_ref[0, :, :D] + acc_ref[1, :, :D] + sout_ref[...]
    s = acc_ref[0, :, D:D + H] + acc_ref[1, :, D:D + H] + ssum_ref[...]
    s128 = jnp.dot(s, _head_selector("hd"), preferred_element_type=jnp.float32)
    hd = tot / (s128 + 1e-16)
    mu = jnp.mean(hd, axis=-1, keepdims=True)
    var = jnp.mean((hd - mu) ** 2, axis=-1, keepdims=True)
    y = (hd - mu) / jnp.sqrt(var + 1e-5) * g_ref[...] + b_ref[...]
    o_ref[...] = jnp.maximum(y, 0.0)


def _sc_edge_body(kv_hbm, q_hbm, et_hbm, src_hbm, dst_hbm, ety_hbm,
                  out_hbm,
                  src_i, ety_i, dst_i, kv_rows, q_rows, e_rows, contrib,
                  acc, sem_kv, sem_q, sem_e):
    c = lax.axis_index("c")
    s = lax.axis_index("s")
    wid = s * NC + c

    zeros16 = jnp.zeros((16,), jnp.float32)
    zvec = ACC_W // 16

    def zfill(i, carry):
        contrib[i // zvec, pl.ds((i % zvec) * 16, 16)] = zeros16
        return carry

    lax.fori_loop(0, ZR * zvec, zfill, 0)

    def zcopy(r, carry):
        pltpu.sync_copy(contrib.at[pl.ds(0, ZR)],
                        acc.at[pl.ds(s * RPT + r * ZR, ZR)])
        return carry

    lax.fori_loop(0, RPT // ZR, zcopy, 0)
    plsc.subcore_barrier()

    lane = lax.broadcasted_iota(jnp.int32, (16,), 0)
    base_w = wid * EPW

    def chunk(ci, carry):
        base = base_w + ci * C
        pltpu.sync_copy(src_hbm.at[pl.ds(base, C)], src_i)
        pltpu.sync_copy(ety_hbm.at[pl.ds(base, C)], ety_i)
        pltpu.sync_copy(dst_hbm.at[pl.ds(base, C)], dst_i.at[0])
        cp_kv = pltpu.async_copy(kv_hbm.at[src_i], kv_rows, sem_kv)
        cp_q = pltpu.async_copy(q_hbm.at[dst_i.at[0]], q_rows, sem_q)
        cp_e = pltpu.async_copy(et_hbm.at[ety_i], e_rows, sem_e)
        cp_kv.wait()
        cp_q.wait()
        cp_e.wait()

        def edge(i, icarry):
            wvec = zeros16
            for h in range(H):
                off = h * DH
                ev = e_rows[i, pl.ds(off, DH)]
                qv = q_rows[i, pl.ds(off, DH)]
                kj = kv_rows[i, pl.ds(off, DH)] + ev
                a = jnp.sum(qv * kj) * 0.25
                wf = jnp.exp(jnp.full((16,), a, jnp.float32))
                vj = kv_rows[i, pl.ds(D + off, DH)] + ev
                contrib[i, pl.ds(off, DH)] = wf * vj
                wvec = jnp.where(lane == h, wf, wvec)
            contrib[i, pl.ds(D, 16)] = wvec
            return icarry

        lax.fori_loop(0, C, edge, 0)
        pltpu.sync_copy(contrib, acc.at[dst_i.at[0]], add=True)
        return carry

    lax.fori_loop(0, NCHUNK, chunk, 0)
    plsc.subcore_barrier()

    pltpu.sync_copy(acc.at[pl.ds(s * RPT, RPT)],
                    out_hbm.at[c, pl.ds(s * RPT, RPT)])


_tc_qkv = pl.pallas_call(
    _tc_qkv_body,
    out_shape=(
        jax.ShapeDtypeStruct((N, 2 * D), jnp.float32),
        jax.ShapeDtypeStruct((N, D), jnp.float32),
        jax.ShapeDtypeStruct((N, D), jnp.float32),
        jax.ShapeDtypeStruct((N, H), jnp.float32),
    ),
)

_tc_combine = pl.pallas_call(
    _tc_combine_body,
    out_shape=jax.ShapeDtypeStruct((N, D), jnp.float32),
)

_sc_edge_pass = pl.kernel(
    _sc_edge_body,
    out_type=jax.ShapeDtypeStruct((NC, N, ACC_W), jnp.float32),
    mesh=plsc.VectorSubcoreMesh(core_axis_name="c", subcore_axis_name="s",
                                num_cores=NC, num_subcores=NS),
    scratch_types=[
        pltpu.VMEM((C,), jnp.int32),
        pltpu.VMEM((C,), jnp.int32),
        pltpu.VMEM((1, C), jnp.int32),
        pltpu.VMEM((C, 2 * D), jnp.float32),
        pltpu.VMEM((C, D), jnp.float32),
        pltpu.VMEM((C, D), jnp.float32),
        pltpu.VMEM((C, ACC_W), jnp.float32),
        pltpu.VMEM_SHARED((N, ACC_W), jnp.float32),
        pltpu.SemaphoreType.DMA,
        pltpu.SemaphoreType.DMA,
        pltpu.SemaphoreType.DMA,
    ],
    compiler_params=pltpu.CompilerParams(use_tc_tiling_on_sc=False,
                                         needs_layout_passes=False),
)


def kernel(x, edge_index, edge_type, Wq0, Wk0, Wv0, Et0, g0, b0,
           Wq1, Wk1, Wv1, Et1, g1, b1):
    src = edge_index[0].astype(jnp.int32)
    dst = edge_index[1].astype(jnp.int32)
    ety = edge_type.astype(jnp.int32)
    h = x
    for Wq, Wk, Wv, Et, g, b in ((Wq0, Wk0, Wv0, Et0, g0, b0),
                                 (Wq1, Wk1, Wv1, Et1, g1, b1)):
        kv, q, sout, ssum = _tc_qkv(h, Wq, Wk, Wv, Et[0:1, :])
        acc = _sc_edge_pass(kv, q, Et, src, dst, ety)
        h = _tc_combine(acc, sout, ssum, g.reshape(1, D), b.reshape(1, D))
    return h


# speedup vs baseline: 16.1105x; 1.1693x over previous
import jax
import jax.numpy as jnp
from jax import lax
from jax.experimental import pallas as pl
from jax.experimental.pallas import tpu as pltpu
from jax.experimental.pallas import tpu_sc as plsc

N = 10000
E = 320000
D = 128
H = 8
DH = 16
NT = 16
NC = 2
NS = 16
NW = NC * NS
EPW = E // NW
C = 40
NCHUNK = EPW // C
SUP = 10
NSUP = NCHUNK // SUP
ACC_W = 144
RPT = N // NS
ZR = 25


def _head_selector(shape_hd):
    if shape_hd == "dh":
        d = lax.broadcasted_iota(jnp.int32, (D, H), 0) // DH
        h = lax.broadcasted_iota(jnp.int32, (D, H), 1)
    else:
        h = lax.broadcasted_iota(jnp.int32, (H, D), 0)
        d = lax.broadcasted_iota(jnp.int32, (H, D), 1) // DH
    return (d == h).astype(jnp.float32)


def _tc_qkv_body(x_ref, wq_ref, wk_ref, wv_ref, e0_ref,
                 kv_ref, q_ref, sout_ref, ssum_ref):
    x = x_ref[...]
    q = jnp.dot(x, wq_ref[...], preferred_element_type=jnp.float32)
    k = jnp.dot(x, wk_ref[...], preferred_element_type=jnp.float32)
    v = jnp.dot(x, wv_ref[...], preferred_element_type=jnp.float32)
    q_ref[...] = q
    kv_ref[:, :D] = k
    kv_ref[:, D:] = v
    e0 = e0_ref[...]
    sel_dh = _head_selector("dh")
    a = jnp.dot(q * (k + e0), sel_dh, preferred_element_type=jnp.float32) * 0.25
    w = jnp.exp(a)
    ssum_ref[...] = w
    w128 = jnp.dot(w, _head_selector("hd"), preferred_element_type=jnp.float32)
    sout_ref[...] = w128 * (v + e0)


def _tc_combine_body(acc_ref, sout_ref, ssum_ref, g_ref, b_ref, o_ref):
    tot = acc_ref[0, :, :D] + acc_ref[1, :, :D] + sout_ref[...]
    s = acc_ref[0, :, D:D + H] + acc_ref[1, :, D:D + H] + ssum_ref[...]
    s128 = jnp.dot(s, _head_selector("hd"), preferred_element_type=jnp.float32)
    hd = tot / (s128 + 1e-16)
    mu = jnp.mean(hd, axis=-1, keepdims=True)
    var = jnp.mean((hd - mu) ** 2, axis=-1, keepdims=True)
    y = (hd - mu) / jnp.sqrt(var + 1e-5) * g_ref[...] + b_ref[...]
    o_ref[...] = jnp.maximum(y, 0.0)


def _sc_edge_body(kv_hbm, q_hbm, et_hbm, src_hbm, dst_hbm, ety_hbm,
                  out_hbm,
                  src_i, dst_i, ety_i, kv0, kv1, q0, q1, contrib, et_buf,
                  acc, skv0, skv1, sq0, sq1):
    c = lax.axis_index("c")
    s = lax.axis_index("s")
    wid = s * NC + c

    pltpu.sync_copy(et_hbm, et_buf)

    zeros16 = jnp.zeros((16,), jnp.float32)
    zvec = ACC_W // 16

    def zfill(i, carry):
        contrib[i // zvec, pl.ds((i % zvec) * 16, 16)] = zeros16
        return carry

    lax.fori_loop(0, ZR * zvec, zfill, 0)

    def zcopy(r, carry):
        pltpu.sync_copy(contrib.at[pl.ds(0, ZR)],
                        acc.at[pl.ds(s * RPT + r * ZR, ZR)])
        return carry

    lax.fori_loop(0, RPT // ZR, zcopy, 0)
    plsc.subcore_barrier()

    lane = lax.broadcasted_iota(jnp.int32, (16,), 0)
    bufs = ((kv0, q0, skv0, sq0), (kv1, q1, skv1, sq1))

    def fire(r, kvb, qb, skv, sq):
        dkv = pltpu.async_copy(kv_hbm.at[src_i.at[r]], kvb, skv)
        dq = pltpu.async_copy(q_hbm.at[dst_i.at[r]], qb, sq)
        return dkv, dq

    def compute_chunk(r, kvb, qb):
        def group(gg, gcarry):
            base = pl.multiple_of(r * C + gg * 8, 8)
            tv = ety_i[pl.ds(base, 16)]
            for j in range(8):
                i = gg * 8 + j
                t = tv[j]
                wvec = zeros16
                for h in range(H):
                    off = h * DH
                    ev = et_buf[t, pl.ds(off, DH)]
                    qv = qb[i, pl.ds(off, DH)]
                    kj = kvb[i, pl.ds(off, DH)] + ev
                    a = jnp.sum(qv * kj) * 0.25
                    wf = jnp.exp(jnp.full((16,), a, jnp.float32))
                    vj = kvb[i, pl.ds(D + off, DH)] + ev
                    contrib[i, pl.ds(off, DH)] = wf * vj
                    wvec = jnp.where(lane == h, wf, wvec)
                contrib[i, pl.ds(D, 16)] = wvec
            return gcarry

        lax.fori_loop(0, C // 8, group, 0)

    def super_body(sp, carry):
        row0 = wid * NCHUNK + sp * SUP
        pltpu.sync_copy(src_hbm.at[pl.ds(row0, SUP)], src_i)
        pltpu.sync_copy(dst_hbm.at[pl.ds(row0, SUP)], dst_i)
        pltpu.sync_copy(ety_hbm.at[pl.ds(row0 * C, SUP * C)],
                        ety_i.at[pl.ds(0, SUP * C)])
        fire(0, *bufs[0])
        fire(1, *bufs[1])

        def pair(t, pcarry):
            for b in range(2):
                r = 2 * t + b
                kvb, qb, skv, sq = bufs[b]
                pltpu.make_async_copy(kv_hbm.at[src_i.at[r]], kvb, skv).wait()
                pltpu.make_async_copy(q_hbm.at[dst_i.at[r]], qb, sq).wait()
                compute_chunk(r, kvb, qb)
                pltpu.sync_copy(contrib, acc.at[dst_i.at[r]], add=True)

                @pl.when(r + 2 < SUP)
                def _():
                    fire(r + 2, kvb, qb, skv, sq)
            return pcarry

        lax.fori_loop(0, SUP // 2, pair, 0)
        return carry

    lax.fori_loop(0, NSUP, super_body, 0)
    plsc.subcore_barrier()

    pltpu.sync_copy(acc.at[pl.ds(s * RPT, RPT)],
                    out_hbm.at[c, pl.ds(s * RPT, RPT)])


_tc_qkv = pl.pallas_call(
    _tc_qkv_body,
    out_shape=(
        jax.ShapeDtypeStruct((N, 2 * D), jnp.float32),
        jax.ShapeDtypeStruct((N, D), jnp.float32),
        jax.ShapeDtypeStruct((N, D), jnp.float32),
        jax.ShapeDtypeStruct((N, H), jnp.float32),
    ),
)

_tc_combine = pl.pallas_call(
    _tc_combine_body,
    out_shape=jax.ShapeDtypeStruct((N, D), jnp.float32),
)

_sc_edge_pass = pl.kernel(
    _sc_edge_body,
    out_type=jax.ShapeDtypeStruct((NC, N, ACC_W), jnp.float32),
    mesh=plsc.VectorSubcoreMesh(core_axis_name="c", subcore_axis_name="s",
                                num_cores=NC, num_subcores=NS),
    scratch_types=[
        pltpu.VMEM((SUP, C), jnp.int32),
        pltpu.VMEM((SUP, C), jnp.int32),
        pltpu.VMEM((SUP * C + 16,), jnp.int32),
        pltpu.VMEM((C, 2 * D), jnp.float32),
        pltpu.VMEM((C, 2 * D), jnp.float32),
        pltpu.VMEM((C, D), jnp.float32),
        pltpu.VMEM((C, D), jnp.float32),
        pltpu.VMEM((C, ACC_W), jnp.float32),
        pltpu.VMEM((NT, D), jnp.float32),
        pltpu.VMEM_SHARED((N, ACC_W), jnp.float32),
        pltpu.SemaphoreType.DMA,
        pltpu.SemaphoreType.DMA,
        pltpu.SemaphoreType.DMA,
        pltpu.SemaphoreType.DMA,
    ],
    compiler_params=pltpu.CompilerParams(use_tc_tiling_on_sc=False,
                                         needs_layout_passes=False),
)


def kernel(x, edge_index, edge_type, Wq0, Wk0, Wv0, Et0, g0, b0,
           Wq1, Wk1, Wv1, Et1, g1, b1):
    src = edge_index[0].astype(jnp.int32).reshape(E // C, C)
    dst = edge_index[1].astype(jnp.int32).reshape(E // C, C)
    ety = edge_type.astype(jnp.int32)
    h = x
    for Wq, Wk, Wv, Et, g, b in ((Wq0, Wk0, Wv0, Et0, g0, b0),
                                 (Wq1, Wk1, Wv1, Et1, g1, b1)):
        kv, q, sout, ssum = _tc_qkv(h, Wq, Wk, Wv, Et[0:1, :])
        acc = _sc_edge_pass(kv, q, Et, src, dst, ety)
        h = _tc_combine(acc, sout, ssum, g.reshape(1, D), b.reshape(1, D))
    return h


# speedup vs baseline: 18.5901x; 1.1539x over previous
import jax
import jax.numpy as jnp
from jax import lax
from jax.experimental import pallas as pl
from jax.experimental.pallas import tpu as pltpu
from jax.experimental.pallas import tpu_sc as plsc

N = 10000
E = 320000
D = 128
H = 8
DH = 16
NT = 16
NC = 2
NS = 16
NW = NC * NS
EPW = E // NW
C = 40
NCHUNK = EPW // C
SUP = 10
NSUP = NCHUNK // SUP
ACC_W = 144
RPT = N // NS
ZR = 25


def _head_selector(shape_hd):
    if shape_hd == "dh":
        d = lax.broadcasted_iota(jnp.int32, (D, H), 0) // DH
        h = lax.broadcasted_iota(jnp.int32, (D, H), 1)
    else:
        h = lax.broadcasted_iota(jnp.int32, (H, D), 0)
        d = lax.broadcasted_iota(jnp.int32, (H, D), 1) // DH
    return (d == h).astype(jnp.float32)


def _tc_qkv_body(x_ref, wq_ref, wk_ref, wv_ref, e0_ref,
                 kv_ref, q_ref, sout_ref, ssum_ref):
    x = x_ref[...]
    q = jnp.dot(x, wq_ref[...], preferred_element_type=jnp.float32)
    k = jnp.dot(x, wk_ref[...], preferred_element_type=jnp.float32)
    v = jnp.dot(x, wv_ref[...], preferred_element_type=jnp.float32)
    q_ref[...] = q * 0.25
    kv_ref[:, :D] = k
    kv_ref[:, D:] = v
    e0 = e0_ref[...]
    sel_dh = _head_selector("dh")
    a = jnp.dot(q * (k + e0), sel_dh, preferred_element_type=jnp.float32) * 0.25
    w = jnp.exp(a)
    ssum_ref[...] = w
    w128 = jnp.dot(w, _head_selector("hd"), preferred_element_type=jnp.float32)
    sout_ref[...] = w128 * (v + e0)


def _tc_combine_body(acc_ref, sout_ref, ssum_ref, g_ref, b_ref, o_ref):
    tot = acc_ref[0, :, :D] + acc_ref[1, :, :D] + sout_ref[...]
    s = acc_ref[0, :, D:D + H] + acc_ref[1, :, D:D + H] + ssum_ref[...]
    s128 = jnp.dot(s, _head_selector("hd"), preferred_element_type=jnp.float32)
    hd = tot / (s128 + 1e-16)
    mu = jnp.mean(hd, axis=-1, keepdims=True)
    var = jnp.mean((hd - mu) ** 2, axis=-1, keepdims=True)
    y = (hd - mu) / jnp.sqrt(var + 1e-5) * g_ref[...] + b_ref[...]
    o_ref[...] = jnp.maximum(y, 0.0)


def _sc_edge_body(kv_hbm, q_hbm, et_hbm, src_hbm, dst_hbm, ety_hbm,
                  out_hbm,
                  src_i, dst_i, ety_i, kv0, kv1, q0, q1, contrib, et_buf,
                  acc, skv0, skv1, sq0, sq1):
    c = lax.axis_index("c")
    s = lax.axis_index("s")
    wid = s * NC + c

    pltpu.sync_copy(et_hbm, et_buf)

    zeros16 = jnp.zeros((16,), jnp.float32)
    zvec = ACC_W // 16

    def zfill(i, carry):
        contrib[i // zvec, pl.ds((i % zvec) * 16, 16)] = zeros16
        return carry

    lax.fori_loop(0, ZR * zvec, zfill, 0)

    def zcopy(r, carry):
        pltpu.sync_copy(contrib.at[pl.ds(0, ZR)],
                        acc.at[pl.ds(s * RPT + r * ZR, ZR)])
        return carry

    lax.fori_loop(0, RPT // ZR, zcopy, 0)
    plsc.subcore_barrier()

    lane = lax.broadcasted_iota(jnp.int32, (16,), 0)
    perms = tuple(lane ^ d for d in (8, 4, 2, 1))
    bufs = ((kv0, q0, skv0, sq0), (kv1, q1, skv1, sq1))

    def fire(r, kvb, qb, skv, sq):
        dkv = pltpu.async_copy(kv_hbm.at[src_i.at[r]], kvb, skv)
        dq = pltpu.async_copy(q_hbm.at[dst_i.at[r]], qb, sq)
        return dkv, dq

    def compute_chunk(r, kvb, qb):
        def group(gg, gcarry):
            base = pl.multiple_of(r * C + gg * 8, 8)
            tv = ety_i[pl.ds(base, 16)]
            for j in range(8):
                i = gg * 8 + j
                t = tv[j]
                wvec = zeros16
                for h in range(H):
                    off = h * DH
                    ev = et_buf[t, pl.ds(off, DH)]
                    qv = qb[i, pl.ds(off, DH)]
                    kj = kvb[i, pl.ds(off, DH)] + ev
                    a = qv * kj
                    for p in perms:
                        a = a + a.at[p].get(mode="promise_in_bounds")
                    wf = jnp.exp(a)
                    vj = kvb[i, pl.ds(D + off, DH)] + ev
                    contrib[i, pl.ds(off, DH)] = wf * vj
                    wvec = jnp.where(lane == h, wf, wvec)
                contrib[i, pl.ds(D, 16)] = wvec
            return gcarry

        lax.fori_loop(0, C // 8, group, 0)

    def super_body(sp, carry):
        row0 = wid * NCHUNK + sp * SUP
        pltpu.sync_copy(src_hbm.at[pl.ds(row0, SUP)], src_i)
        pltpu.sync_copy(dst_hbm.at[pl.ds(row0, SUP)], dst_i)
        pltpu.sync_copy(ety_hbm.at[pl.ds(row0 * C, SUP * C)],
                        ety_i.at[pl.ds(0, SUP * C)])
        fire(0, *bufs[0])
        fire(1, *bufs[1])

        def pair(t, pcarry):
            for b in range(2):
                r = 2 * t + b
                kvb, qb, skv, sq = bufs[b]
                pltpu.make_async_copy(kv_hbm.at[src_i.at[r]], kvb, skv).wait()
                pltpu.make_async_copy(q_hbm.at[dst_i.at[r]], qb, sq).wait()
                compute_chunk(r, kvb, qb)
                pltpu.sync_copy(contrib, acc.at[dst_i.at[r]], add=True)

                @pl.when(r + 2 < SUP)
                def _():
                    fire(r + 2, kvb, qb, skv, sq)
            return pcarry

        lax.fori_loop(0, SUP // 2, pair, 0)
        return carry

    lax.fori_loop(0, NSUP, super_body, 0)
    plsc.subcore_barrier()

    pltpu.sync_copy(acc.at[pl.ds(s * RPT, RPT)],
                    out_hbm.at[c, pl.ds(s * RPT, RPT)])


_tc_qkv = pl.pallas_call(
    _tc_qkv_body,
    out_shape=(
        jax.ShapeDtypeStruct((N, 2 * D), jnp.float32),
        jax.ShapeDtypeStruct((N, D), jnp.float32),
        jax.ShapeDtypeStruct((N, D), jnp.float32),
        jax.ShapeDtypeStruct((N, H), jnp.float32),
    ),
)

_tc_combine = pl.pallas_call(
    _tc_combine_body,
    out_shape=jax.ShapeDtypeStruct((N, D), jnp.float32),
)

_sc_edge_pass = pl.kernel(
    _sc_edge_body,
    out_type=jax.ShapeDtypeStruct((NC, N, ACC_W), jnp.float32),
    mesh=plsc.VectorSubcoreMesh(core_axis_name="c", subcore_axis_name="s",
                                num_cores=NC, num_subcores=NS),
    scratch_types=[
        pltpu.VMEM((SUP, C), jnp.int32),
        pltpu.VMEM((SUP, C), jnp.int32),
        pltpu.VMEM((SUP * C + 16,), jnp.int32),
        pltpu.VMEM((C, 2 * D), jnp.float32),
        pltpu.VMEM((C, 2 * D), jnp.float32),
        pltpu.VMEM((C, D), jnp.float32),
        pltpu.VMEM((C, D), jnp.float32),
        pltpu.VMEM((C, ACC_W), jnp.float32),
        pltpu.VMEM((NT, D), jnp.float32),
        pltpu.VMEM_SHARED((N, ACC_W), jnp.float32),
        pltpu.SemaphoreType.DMA,
        pltpu.SemaphoreType.DMA,
        pltpu.SemaphoreType.DMA,
        pltpu.SemaphoreType.DMA,
    ],
    compiler_params=pltpu.CompilerParams(use_tc_tiling_on_sc=False,
                                         needs_layout_passes=False),
)


def kernel(x, edge_index, edge_type, Wq0, Wk0, Wv0, Et0, g0, b0,
           Wq1, Wk1, Wv1, Et1, g1, b1):
    src = edge_index[0].astype(jnp.int32).reshape(E // C, C)
    dst = edge_index[1].astype(jnp.int32).reshape(E // C, C)
    ety = edge_type.astype(jnp.int32)
    h = x
    for Wq, Wk, Wv, Et, g, b in ((Wq0, Wk0, Wv0, Et0, g0, b0),
                                 (Wq1, Wk1, Wv1, Et1, g1, b1)):
        kv, q, sout, ssum = _tc_qkv(h, Wq, Wk, Wv, Et[0:1, :])
        acc = _sc_edge_pass(kv, q, Et, src, dst, ety)
        h = _tc_combine(acc, sout, ssum, g.reshape(1, D), b.reshape(1, D))
    return h


# speedup vs baseline: 51.9795x; 2.7961x over previous
import jax
import jax.numpy as jnp
from jax import lax
from jax.experimental import pallas as pl
from jax.experimental.pallas import tpu as pltpu
from jax.experimental.pallas import tpu_sc as plsc

N = 10000
E = 320000
D = 128
H = 8
DH = 16
NT = 16
NC = 2
NS = 16
NW = NC * NS
EPW = E // NW
C = 40
NCHUNK = EPW // C
SUP = 10
NSUP = NCHUNK // SUP
ACC_W = 144
RPT = N // NS
ZR = 25


def _head_selector(shape_hd):
    if shape_hd == "dh":
        d = lax.broadcasted_iota(jnp.int32, (D, H), 0) // DH
        h = lax.broadcasted_iota(jnp.int32, (D, H), 1)
    else:
        h = lax.broadcasted_iota(jnp.int32, (H, D), 0)
        d = lax.broadcasted_iota(jnp.int32, (H, D), 1) // DH
    return (d == h).astype(jnp.float32)


def _tc_qkv_body(x_ref, wq_ref, wk_ref, wv_ref, e0_ref,
                 kv_ref, q_ref, sout_ref, ssum_ref):
    x = x_ref[...]
    q = jnp.dot(x, wq_ref[...], preferred_element_type=jnp.float32)
    k = jnp.dot(x, wk_ref[...], preferred_element_type=jnp.float32)
    v = jnp.dot(x, wv_ref[...], preferred_element_type=jnp.float32)
    q_ref[...] = q * 0.25
    kv_ref[:, :D] = k
    kv_ref[:, D:] = v
    e0 = e0_ref[...]
    sel_dh = _head_selector("dh")
    a = jnp.dot(q * (k + e0), sel_dh, preferred_element_type=jnp.float32) * 0.25
    w = jnp.exp(a)
    ssum_ref[...] = w
    w128 = jnp.dot(w, _head_selector("hd"), preferred_element_type=jnp.float32)
    sout_ref[...] = w128 * (v + e0)


def _tc_combine_body(acc_ref, sout_ref, ssum_ref, g_ref, b_ref, o_ref):
    tot = acc_ref[0, :, :D] + acc_ref[1, :, :D] + sout_ref[...]
    s = acc_ref[0, :, D:D + H] + acc_ref[1, :, D:D + H] + ssum_ref[...]
    s128 = jnp.dot(s, _head_selector("hd"), preferred_element_type=jnp.float32)
    hd = tot / (s128 + 1e-16)
    mu = jnp.mean(hd, axis=-1, keepdims=True)
    var = jnp.mean((hd - mu) ** 2, axis=-1, keepdims=True)
    y = (hd - mu) / jnp.sqrt(var + 1e-5) * g_ref[...] + b_ref[...]
    o_ref[...] = jnp.maximum(y, 0.0)


def _sc_edge_body(kv_hbm, q_hbm, et_hbm, src_hbm, dst_hbm, ety_hbm,
                  out_hbm,
                  src_i, dst_i, ety_i, kv0, kv1, q0, q1, contrib, et_buf,
                  acc, skv0, skv1, sq0, sq1):
    c = lax.axis_index("c")
    s = lax.axis_index("s")
    wid = s * NC + c

    pltpu.sync_copy(et_hbm, et_buf)

    zeros16 = jnp.zeros((16,), jnp.float32)
    zvec = ACC_W // 16

    def zfill(i, carry):
        contrib[i // zvec, pl.ds((i % zvec) * 16, 16)] = zeros16
        return carry

    lax.fori_loop(0, ZR * zvec, zfill, 0)

    def zcopy(r, carry):
        pltpu.sync_copy(contrib.at[pl.ds(0, ZR)],
                        acc.at[pl.ds(s * RPT + r * ZR, ZR)])
        return carry

    lax.fori_loop(0, RPT // ZR, zcopy, 0)
    plsc.subcore_barrier()

    lane = lax.broadcasted_iota(jnp.int32, (16,), 0)
    perms = tuple(lane ^ d for d in (8, 4, 2, 1))
    bufs = ((kv0, q0, skv0, sq0), (kv1, q1, skv1, sq1))

    def fire(r, kvb, qb, skv, sq):
        dkv = pltpu.async_copy(kv_hbm.at[src_i.at[r]], kvb, skv)
        dq = pltpu.async_copy(q_hbm.at[dst_i.at[r]], qb, sq)
        return dkv, dq

    def compute_chunk(r, kvb, qb):
        def group(gg, gcarry):
            base = pl.multiple_of(r * C + gg * 8, 8)
            tv = ety_i[pl.ds(base, 16)]
            for j in range(8):
                i = gg * 8 + j
                t = tv[j]
                wvec = zeros16
                for hh in (0, 4):
                    hs = tuple(range(hh, hh + 4))
                    offs = [h * DH for h in hs]
                    evs = [et_buf[t, pl.ds(o, DH)] for o in offs]
                    qvs = [qb[i, pl.ds(o, DH)] for o in offs]
                    kjs = [kvb[i, pl.ds(o, DH)] for o in offs]
                    vjs = [kvb[i, pl.ds(D + o, DH)] for o in offs]
                    aa = [(k + e) * q for k, e, q in zip(kjs, evs, qvs)]
                    for p in perms:
                        aa = [a + a.at[p].get(mode="promise_in_bounds")
                              for a in aa]
                    wfs = [jnp.exp(a) for a in aa]
                    outs = [w * (v + e) for w, v, e in zip(wfs, vjs, evs)]
                    for h, o, w, ov in zip(hs, offs, wfs, outs):
                        contrib[i, pl.ds(o, DH)] = ov
                        wvec = jnp.where(lane == h, w, wvec)
                contrib[i, pl.ds(D, 16)] = wvec
            return gcarry

        lax.fori_loop(0, C // 8, group, 0)

    def super_body(sp, carry):
        row0 = wid * NCHUNK + sp * SUP
        pltpu.sync_copy(src_hbm.at[pl.ds(row0, SUP)], src_i)
        pltpu.sync_copy(dst_hbm.at[pl.ds(row0, SUP)], dst_i)
        pltpu.sync_copy(ety_hbm.at[pl.ds(row0 * C, SUP * C)],
                        ety_i.at[pl.ds(0, SUP * C)])
        fire(0, *bufs[0])
        fire(1, *bufs[1])

        def pair(t, pcarry):
            for b in range(2):
                r = 2 * t + b
                kvb, qb, skv, sq = bufs[b]
                pltpu.make_async_copy(kv_hbm.at[src_i.at[r]], kvb, skv).wait()
                pltpu.make_async_copy(q_hbm.at[dst_i.at[r]], qb, sq).wait()
                compute_chunk(r, kvb, qb)
                pltpu.sync_copy(contrib, acc.at[dst_i.at[r]], add=True)

                @pl.when(r + 2 < SUP)
                def _():
                    fire(r + 2, kvb, qb, skv, sq)
            return pcarry

        lax.fori_loop(0, SUP // 2, pair, 0)
        return carry

    lax.fori_loop(0, NSUP, super_body, 0)
    plsc.subcore_barrier()

    pltpu.sync_copy(acc.at[pl.ds(s * RPT, RPT)],
                    out_hbm.at[c, pl.ds(s * RPT, RPT)])


_tc_qkv = pl.pallas_call(
    _tc_qkv_body,
    out_shape=(
        jax.ShapeDtypeStruct((N, 2 * D), jnp.float32),
        jax.ShapeDtypeStruct((N, D), jnp.float32),
        jax.ShapeDtypeStruct((N, D), jnp.float32),
        jax.ShapeDtypeStruct((N, H), jnp.float32),
    ),
)

_tc_combine = pl.pallas_call(
    _tc_combine_body,
    out_shape=jax.ShapeDtypeStruct((N, D), jnp.float32),
)

_sc_edge_pass = pl.kernel(
    _sc_edge_body,
    out_type=jax.ShapeDtypeStruct((NC, N, ACC_W), jnp.float32),
    mesh=plsc.VectorSubcoreMesh(core_axis_name="c", subcore_axis_name="s",
                                num_cores=NC, num_subcores=NS),
    scratch_types=[
        pltpu.VMEM((SUP, C), jnp.int32),
        pltpu.VMEM((SUP, C), jnp.int32),
        pltpu.VMEM((SUP * C + 16,), jnp.int32),
        pltpu.VMEM((C, 2 * D), jnp.float32),
        pltpu.VMEM((C, 2 * D), jnp.float32),
        pltpu.VMEM((C, D), jnp.float32),
        pltpu.VMEM((C, D), jnp.float32),
        pltpu.VMEM((C, ACC_W), jnp.float32),
        pltpu.VMEM((NT, D), jnp.float32),
        pltpu.VMEM_SHARED((N, ACC_W), jnp.float32),
        pltpu.SemaphoreType.DMA,
        pltpu.SemaphoreType.DMA,
        pltpu.SemaphoreType.DMA,
        pltpu.SemaphoreType.DMA,
    ],
    compiler_params=pltpu.CompilerParams(use_tc_tiling_on_sc=False,
                                         needs_layout_passes=False),
)


def kernel(x, edge_index, edge_type, Wq0, Wk0, Wv0, Et0, g0, b0,
           Wq1, Wk1, Wv1, Et1, g1, b1):
    src = edge_index[0].astype(jnp.int32).reshape(E // C, C)
    dst = edge_index[1].astype(jnp.int32).reshape(E // C, C)
    ety = edge_type.astype(jnp.int32)
    h = x
    for Wq, Wk, Wv, Et, g, b in ((Wq0, Wk0, Wv0, Et0, g0, b0),
                                 (Wq1, Wk1, Wv1, Et1, g1, b1)):
        kv, q, sout, ssum = _tc_qkv(h, Wq, Wk, Wv, Et[0:1, :])
        acc = _sc_edge_pass(kv, q, Et, src, dst, ety)
        h = _tc_combine(acc, sout, ssum, g.reshape(1, D), b.reshape(1, D))
    return h


# speedup vs baseline: 78.3337x; 1.5070x over previous
import jax
import jax.numpy as jnp
from jax import lax
from jax.experimental import pallas as pl
from jax.experimental.pallas import tpu as pltpu
from jax.experimental.pallas import tpu_sc as plsc

N = 10000
E = 320000
D = 128
H = 8
DH = 16
NT = 16
NC = 2
NS = 16
NW = NC * NS
EPW = E // NW
C = 40
NCHUNK = EPW // C
SUP = 10
NSUP = NCHUNK // SUP
ACC_W = 144
RPT = N // NS
ZR = 25


def _head_selector(shape_hd):
    if shape_hd == "dh":
        d = lax.broadcasted_iota(jnp.int32, (D, H), 0) // DH
        h = lax.broadcasted_iota(jnp.int32, (D, H), 1)
    else:
        h = lax.broadcasted_iota(jnp.int32, (H, D), 0)
        d = lax.broadcasted_iota(jnp.int32, (H, D), 1) // DH
    return (d == h).astype(jnp.float32)


def _tc_qkv_body(x_ref, wq_ref, wk_ref, wv_ref, e0_ref,
                 kv_ref, q_ref, sout_ref, ssum_ref):
    x = x_ref[...]
    q = jnp.dot(x, wq_ref[...], preferred_element_type=jnp.float32)
    k = jnp.dot(x, wk_ref[...], preferred_element_type=jnp.float32)
    v = jnp.dot(x, wv_ref[...], preferred_element_type=jnp.float32)
    q_ref[...] = q * 0.25
    kv_ref[:, :D] = k
    kv_ref[:, D:] = v
    e0 = e0_ref[...]
    sel_dh = _head_selector("dh")
    a = jnp.dot(q * (k + e0), sel_dh, preferred_element_type=jnp.float32) * 0.25
    w = jnp.exp(a)
    ssum_ref[...] = w
    w128 = jnp.dot(w, _head_selector("hd"), preferred_element_type=jnp.float32)
    sout_ref[...] = w128 * (v + e0)


def _tc_combine_body(acc_ref, sout_ref, ssum_ref, g_ref, b_ref, o_ref):
    tot = acc_ref[0, :, :D] + acc_ref[1, :, :D] + sout_ref[...]
    s = acc_ref[0, :, D:D + H] + acc_ref[1, :, D:D + H] + ssum_ref[...]
    s128 = jnp.dot(s, _head_selector("hd"), preferred_element_type=jnp.float32)
    hd = tot / (s128 + 1e-16)
    mu = jnp.mean(hd, axis=-1, keepdims=True)
    var = jnp.mean((hd - mu) ** 2, axis=-1, keepdims=True)
    y = (hd - mu) / jnp.sqrt(var + 1e-5) * g_ref[...] + b_ref[...]
    o_ref[...] = jnp.maximum(y, 0.0)


def _sc_edge_body(kv_hbm, q_hbm, et_hbm, src_hbm, dst_hbm, ety_hbm,
                  out_hbm,
                  src_i, dst_i, ety_i, kv0, kv1, q0, q1, contrib, et_buf,
                  acc, skv0, skv1, sq0, sq1):
    c = lax.axis_index("c")
    s = lax.axis_index("s")
    wid = s * NC + c

    pltpu.sync_copy(et_hbm, et_buf)

    zeros16 = jnp.zeros((16,), jnp.float32)
    zvec = ACC_W // 16

    def zfill(i, carry):
        contrib[i // zvec, pl.ds((i % zvec) * 16, 16)] = zeros16
        return carry

    lax.fori_loop(0, ZR * zvec, zfill, 0)

    def zcopy(r, carry):
        pltpu.sync_copy(contrib.at[pl.ds(0, ZR)],
                        acc.at[pl.ds(s * RPT + r * ZR, ZR)])
        return carry

    lax.fori_loop(0, RPT // ZR, zcopy, 0)
    plsc.subcore_barrier()

    lane = lax.broadcasted_iota(jnp.int32, (16,), 0)
    last = jnp.full((16,), 15, jnp.int32)
    bufs = ((kv0, q0, skv0, sq0), (kv1, q1, skv1, sq1))

    def fire(r, kvb, qb, skv, sq):
        dkv = pltpu.async_copy(kv_hbm.at[src_i.at[r]], kvb, skv)
        dq = pltpu.async_copy(q_hbm.at[dst_i.at[r]], qb, sq)
        return dkv, dq

    def compute_chunk(r, kvb, qb):
        def group(gg, gcarry):
            base = pl.multiple_of(r * C + gg * 8, 8)
            tv = ety_i[pl.ds(base, 16)]
            for j in range(8):
                i = gg * 8 + j
                t = tv[j]
                wvec = zeros16
                hs = tuple(range(H))
                offs = [h * DH for h in hs]
                evs = [et_buf[t, pl.ds(o, DH)] for o in offs]
                qvs = [qb[i, pl.ds(o, DH)] for o in offs]
                kjs = [kvb[i, pl.ds(o, DH)] for o in offs]
                vjs = [kvb[i, pl.ds(D + o, DH)] for o in offs]
                aa = [(k + e) * q for k, e, q in zip(kjs, evs, qvs)]
                aa = [plsc.cumsum(a) for a in aa]
                aa = [a.at[last].get(mode="promise_in_bounds") for a in aa]
                wfs = [jnp.exp(a) for a in aa]
                outs = [w * (v + e) for w, v, e in zip(wfs, vjs, evs)]
                for h, o, w, ov in zip(hs, offs, wfs, outs):
                    contrib[i, pl.ds(o, DH)] = ov
                    wvec = jnp.where(lane == h, w, wvec)
                contrib[i, pl.ds(D, 16)] = wvec
            return gcarry

        lax.fori_loop(0, C // 8, group, 0)

    def super_body(sp, carry):
        row0 = wid * NCHUNK + sp * SUP
        pltpu.sync_copy(src_hbm.at[pl.ds(row0, SUP)], src_i)
        pltpu.sync_copy(dst_hbm.at[pl.ds(row0, SUP)], dst_i)
        pltpu.sync_copy(ety_hbm.at[pl.ds(row0 * C, SUP * C)],
                        ety_i.at[pl.ds(0, SUP * C)])
        fire(0, *bufs[0])
        fire(1, *bufs[1])

        def pair(t, pcarry):
            for b in range(2):
                r = 2 * t + b
                kvb, qb, skv, sq = bufs[b]
                pltpu.make_async_copy(kv_hbm.at[src_i.at[r]], kvb, skv).wait()
                pltpu.make_async_copy(q_hbm.at[dst_i.at[r]], qb, sq).wait()
                compute_chunk(r, kvb, qb)
                pltpu.sync_copy(contrib, acc.at[dst_i.at[r]], add=True)

                @pl.when(r + 2 < SUP)
                def _():
                    fire(r + 2, kvb, qb, skv, sq)
            return pcarry

        lax.fori_loop(0, SUP // 2, pair, 0)
        return carry

    lax.fori_loop(0, NSUP, super_body, 0)
    plsc.subcore_barrier()

    pltpu.sync_copy(acc.at[pl.ds(s * RPT, RPT)],
                    out_hbm.at[c, pl.ds(s * RPT, RPT)])


_tc_qkv = pl.pallas_call(
    _tc_qkv_body,
    out_shape=(
        jax.ShapeDtypeStruct((N, 2 * D), jnp.float32),
        jax.ShapeDtypeStruct((N, D), jnp.float32),
        jax.ShapeDtypeStruct((N, D), jnp.float32),
        jax.ShapeDtypeStruct((N, H), jnp.float32),
    ),
)

_tc_combine = pl.pallas_call(
    _tc_combine_body,
    out_shape=jax.ShapeDtypeStruct((N, D), jnp.float32),
)

_sc_edge_pass = pl.kernel(
    _sc_edge_body,
    out_type=jax.ShapeDtypeStruct((NC, N, ACC_W), jnp.float32),
    mesh=plsc.VectorSubcoreMesh(core_axis_name="c", subcore_axis_name="s",
                                num_cores=NC, num_subcores=NS),
    scratch_types=[
        pltpu.VMEM((SUP, C), jnp.int32),
        pltpu.VMEM((SUP, C), jnp.int32),
        pltpu.VMEM((SUP * C + 16,), jnp.int32),
        pltpu.VMEM((C, 2 * D), jnp.float32),
        pltpu.VMEM((C, 2 * D), jnp.float32),
        pltpu.VMEM((C, D), jnp.float32),
        pltpu.VMEM((C, D), jnp.float32),
        pltpu.VMEM((C, ACC_W), jnp.float32),
        pltpu.VMEM((NT, D), jnp.float32),
        pltpu.VMEM_SHARED((N, ACC_W), jnp.float32),
        pltpu.SemaphoreType.DMA,
        pltpu.SemaphoreType.DMA,
        pltpu.SemaphoreType.DMA,
        pltpu.SemaphoreType.DMA,
    ],
    compiler_params=pltpu.CompilerParams(use_tc_tiling_on_sc=False,
                                         needs_layout_passes=False),
)


def kernel(x, edge_index, edge_type, Wq0, Wk0, Wv0, Et0, g0, b0,
           Wq1, Wk1, Wv1, Et1, g1, b1):
    src = edge_index[0].astype(jnp.int32).reshape(E // C, C)
    dst = edge_index[1].astype(jnp.int32).reshape(E // C, C)
    ety = edge_type.astype(jnp.int32)
    h = x
    for Wq, Wk, Wv, Et, g, b in ((Wq0, Wk0, Wv0, Et0, g0, b0),
                                 (Wq1, Wk1, Wv1, Et1, g1, b1)):
        kv, q, sout, ssum = _tc_qkv(h, Wq, Wk, Wv, Et[0:1, :])
        acc = _sc_edge_pass(kv, q, Et, src, dst, ety)
        h = _tc_combine(acc, sout, ssum, g.reshape(1, D), b.reshape(1, D))
    return h


# speedup vs baseline: 81.3030x; 1.0379x over previous
import jax
import jax.numpy as jnp
from jax import lax
from jax.experimental import pallas as pl
from jax.experimental.pallas import tpu as pltpu
from jax.experimental.pallas import tpu_sc as plsc

N = 10000
E = 320000
D = 128
H = 8
DH = 16
NT = 16
NC = 2
NS = 16
NW = NC * NS
EPW = E // NW
C = 40
NCHUNK = EPW // C
SUP = 10
NSUP = NCHUNK // SUP
ACC_W = 144
RPT = N // NS
ZR = 25


def _head_selector(shape_hd):
    if shape_hd == "dh":
        d = lax.broadcasted_iota(jnp.int32, (D, H), 0) // DH
        h = lax.broadcasted_iota(jnp.int32, (D, H), 1)
    else:
        h = lax.broadcasted_iota(jnp.int32, (H, D), 0)
        d = lax.broadcasted_iota(jnp.int32, (H, D), 1) // DH
    return (d == h).astype(jnp.float32)


def _tc_qkv_body(x_ref, wq_ref, wk_ref, wv_ref, e0_ref,
                 kv_ref, q_ref, sout_ref, ssum_ref):
    x = x_ref[...]
    q = jnp.dot(x, wq_ref[...], preferred_element_type=jnp.float32)
    k = jnp.dot(x, wk_ref[...], preferred_element_type=jnp.float32)
    v = jnp.dot(x, wv_ref[...], preferred_element_type=jnp.float32)
    q_ref[...] = q * 0.25
    ss = lax.broadcasted_iota(jnp.int32, (D, D), 0)
    dd = lax.broadcasted_iota(jnp.int32, (D, D), 1)
    src_col = (2 * (dd // 32) + dd % 2) * DH + (dd % 32) // 2
    perm = (ss == src_col).astype(jnp.float32)
    kv_ref[:, :D] = jnp.dot(k, perm,
                            preferred_element_type=jnp.float32).astype(jnp.bfloat16)
    kv_ref[:, D:] = jnp.dot(v, perm,
                            preferred_element_type=jnp.float32).astype(jnp.bfloat16)
    e0 = e0_ref[...]
    sel_dh = _head_selector("dh")
    a = jnp.dot(q * (k + e0), sel_dh, preferred_element_type=jnp.float32) * 0.25
    w = jnp.exp(a)
    ssum_ref[...] = w
    w128 = jnp.dot(w, _head_selector("hd"), preferred_element_type=jnp.float32)
    sout_ref[...] = w128 * (v + e0)


def _tc_combine_body(acc_ref, sout_ref, ssum_ref, g_ref, b_ref, o_ref):
    tot = acc_ref[0, :, :D] + acc_ref[1, :, :D] + sout_ref[...]
    s = acc_ref[0, :, D:D + H] + acc_ref[1, :, D:D + H] + ssum_ref[...]
    s128 = jnp.dot(s, _head_selector("hd"), preferred_element_type=jnp.float32)
    hd = tot / (s128 + 1e-16)
    mu = jnp.mean(hd, axis=-1, keepdims=True)
    var = jnp.mean((hd - mu) ** 2, axis=-1, keepdims=True)
    y = (hd - mu) / jnp.sqrt(var + 1e-5) * g_ref[...] + b_ref[...]
    o_ref[...] = jnp.maximum(y, 0.0)


def _sc_edge_body(kv_hbm, q_hbm, et_hbm, src_hbm, dst_hbm, ety_hbm,
                  out_hbm,
                  src_i, dst_i, ety_i, kv0, kv1, q0, q1, contrib, et_buf,
                  acc, skv0, skv1, sq0, sq1):
    c = lax.axis_index("c")
    s = lax.axis_index("s")
    wid = s * NC + c

    pltpu.sync_copy(et_hbm, et_buf)

    zeros16 = jnp.zeros((16,), jnp.float32)
    zvec = ACC_W // 16

    def zfill(i, carry):
        contrib[i // zvec, pl.ds((i % zvec) * 16, 16)] = zeros16
        return carry

    lax.fori_loop(0, ZR * zvec, zfill, 0)

    def zcopy(r, carry):
        pltpu.sync_copy(contrib.at[pl.ds(0, ZR)],
                        acc.at[pl.ds(s * RPT + r * ZR, ZR)])
        return carry

    lax.fori_loop(0, RPT // ZR, zcopy, 0)
    plsc.subcore_barrier()

    lane = lax.broadcasted_iota(jnp.int32, (16,), 0)
    last = jnp.full((16,), 15, jnp.int32)
    bufs = ((kv0, q0, skv0, sq0), (kv1, q1, skv1, sq1))

    def fire(r, kvb, qb, skv, sq):
        dkv = pltpu.async_copy(kv_hbm.at[src_i.at[r]], kvb, skv)
        dq = pltpu.async_copy(q_hbm.at[dst_i.at[r]], qb, sq)
        return dkv, dq

    def compute_chunk(r, kvb, qb):
        def group(gg, gcarry):
            base = pl.multiple_of(r * C + gg * 8, 8)
            tv = ety_i[pl.ds(base, 16)]
            for j in range(8):
                i = gg * 8 + j
                t = tv[j]
                wvec = zeros16
                hs = tuple(range(H))
                offs = [h * DH for h in hs]
                evs = [et_buf[t, pl.ds(o, DH)] for o in offs]
                qvs = [qb[i, pl.ds(o, DH)] for o in offs]
                kprs = [plsc.unpack(kvb[i, pl.ds(32 * m, 32)],
                                    format=plsc.PackFormat.INTERLEAVED,
                                    preferred_element_type=jnp.float32)
                        for m in range(H // 2)]
                vprs = [plsc.unpack(kvb[i, pl.ds(D + 32 * m, 32)],
                                    format=plsc.PackFormat.INTERLEAVED,
                                    preferred_element_type=jnp.float32)
                        for m in range(H // 2)]
                kjs = [x for pr in kprs for x in pr]
                vjs = [x for pr in vprs for x in pr]
                aa = [(k + e) * q for k, e, q in zip(kjs, evs, qvs)]
                aa = [plsc.cumsum(a) for a in aa]
                aa = [a.at[last].get(mode="promise_in_bounds") for a in aa]
                wfs = [jnp.exp(a) for a in aa]
                outs = [w * (v + e) for w, v, e in zip(wfs, vjs, evs)]
                for h, o, w, ov in zip(hs, offs, wfs, outs):
                    contrib[i, pl.ds(o, DH)] = ov
                    wvec = jnp.where(lane == h, w, wvec)
                contrib[i, pl.ds(D, 16)] = wvec
            return gcarry

        lax.fori_loop(0, C // 8, group, 0)

    def super_body(sp, carry):
        row0 = wid * NCHUNK + sp * SUP
        pltpu.sync_copy(src_hbm.at[pl.ds(row0, SUP)], src_i)
        pltpu.sync_copy(dst_hbm.at[pl.ds(row0, SUP)], dst_i)
        pltpu.sync_copy(ety_hbm.at[pl.ds(row0 * C, SUP * C)],
                        ety_i.at[pl.ds(0, SUP * C)])
        fire(0, *bufs[0])
        fire(1, *bufs[1])

        def pair(t, pcarry):
            for b in range(2):
                r = 2 * t + b
                kvb, qb, skv, sq = bufs[b]
                pltpu.make_async_copy(kv_hbm.at[src_i.at[r]], kvb, skv).wait()
                pltpu.make_async_copy(q_hbm.at[dst_i.at[r]], qb, sq).wait()
                compute_chunk(r, kvb, qb)
                pltpu.sync_copy(contrib, acc.at[dst_i.at[r]], add=True)

                @pl.when(r + 2 < SUP)
                def _():
                    fire(r + 2, kvb, qb, skv, sq)
            return pcarry

        lax.fori_loop(0, SUP // 2, pair, 0)
        return carry

    lax.fori_loop(0, NSUP, super_body, 0)
    plsc.subcore_barrier()

    pltpu.sync_copy(acc.at[pl.ds(s * RPT, RPT)],
                    out_hbm.at[c, pl.ds(s * RPT, RPT)])


_tc_qkv = pl.pallas_call(
    _tc_qkv_body,
    out_shape=(
        jax.ShapeDtypeStruct((N, 2 * D), jnp.bfloat16),
        jax.ShapeDtypeStruct((N, D), jnp.float32),
        jax.ShapeDtypeStruct((N, D), jnp.float32),
        jax.ShapeDtypeStruct((N, H), jnp.float32),
    ),
)

_tc_combine = pl.pallas_call(
    _tc_combine_body,
    out_shape=jax.ShapeDtypeStruct((N, D), jnp.float32),
)

_sc_edge_pass = pl.kernel(
    _sc_edge_body,
    out_type=jax.ShapeDtypeStruct((NC, N, ACC_W), jnp.float32),
    mesh=plsc.VectorSubcoreMesh(core_axis_name="c", subcore_axis_name="s",
                                num_cores=NC, num_subcores=NS),
    scratch_types=[
        pltpu.VMEM((SUP, C), jnp.int32),
        pltpu.VMEM((SUP, C), jnp.int32),
        pltpu.VMEM((SUP * C + 16,), jnp.int32),
        pltpu.VMEM((C, 2 * D), jnp.bfloat16),
        pltpu.VMEM((C, 2 * D), jnp.bfloat16),
        pltpu.VMEM((C, D), jnp.float32),
        pltpu.VMEM((C, D), jnp.float32),
        pltpu.VMEM((C, ACC_W), jnp.float32),
        pltpu.VMEM((NT, D), jnp.float32),
        pltpu.VMEM_SHARED((N, ACC_W), jnp.float32),
        pltpu.SemaphoreType.DMA,
        pltpu.SemaphoreType.DMA,
        pltpu.SemaphoreType.DMA,
        pltpu.SemaphoreType.DMA,
    ],
    compiler_params=pltpu.CompilerParams(use_tc_tiling_on_sc=False,
                                         needs_layout_passes=False),
)


def kernel(x, edge_index, edge_type, Wq0, Wk0, Wv0, Et0, g0, b0,
           Wq1, Wk1, Wv1, Et1, g1, b1):
    src = edge_index[0].astype(jnp.int32).reshape(E // C, C)
    dst = edge_index[1].astype(jnp.int32).reshape(E // C, C)
    ety = edge_type.astype(jnp.int32)
    h = x
    for Wq, Wk, Wv, Et, g, b in ((Wq0, Wk0, Wv0, Et0, g0, b0),
                                 (Wq1, Wk1, Wv1, Et1, g1, b1)):
        kv, q, sout, ssum = _tc_qkv(h, Wq, Wk, Wv, Et[0:1, :])
        acc = _sc_edge_pass(kv, q, Et, src, dst, ety)
        h = _tc_combine(acc, sout, ssum, g.reshape(1, D), b.reshape(1, D))
    return h
